# baseline (speedup 1.0000x reference)
import functools

import jax
import jax.numpy as jnp
from jax import lax
from jax.experimental import pallas as pl
from jax.experimental.pallas import tpu as pltpu
from jax.experimental.pallas import tpu_sc as plsc

_B, _N, _D = 4, 4096, 3
_K = 16
_L = 16
_NC, _NS = 2, 16
_NW = _NC * _NS
_QPW = _B * _N // _NW
_WPB = _NW // _B
_NCHUNK = _N // _L
_QI = 2


def _knn_body(posT_hbm, out_hbm, x_v, y_v, z_v, ctx_v,
              bufda_v, bufia_v, bufdb_v, bufib_v, d2a_v, d2b_v):
    cid = lax.axis_index("c")
    sid = lax.axis_index("s")
    wid = cid * _NS + sid
    b = wid // _WPB
    qoff = (wid % _WPB) * _QPW
    pltpu.sync_copy(posT_hbm.at[pl.ds((b * 3 + 0) * _N, _N)], x_v)
    pltpu.sync_copy(posT_hbm.at[pl.ds((b * 3 + 1) * _N, _N)], y_v)
    pltpu.sync_copy(posT_hbm.at[pl.ds((b * 3 + 2) * _N, _N)], z_v)

    lanes = lax.iota(jnp.int32, _L)

    inf_v = jnp.full((_L,), jnp.inf, jnp.float32)
    zero_i = jnp.zeros((_L,), jnp.int32)

    def _mergev(bd, bi, cd, ci):
        sd, si = plsc.sort_key_val(cd, ci)
        rd = lax.rev(sd, (0,))
        ri = lax.rev(si, (0,))
        keep = bd <= rd
        nd = jnp.where(keep, bd, rd)
        ni = jnp.where(keep, bi, ri)
        nd2, ni2 = plsc.sort_key_val(nd, ni)
        return nd2, ni2

    def q_body(qi, carry):
        qa = qoff + 2 * qi
        qb = qa + 1
        qidxa = zero_i + qa
        qidxb = zero_i + qb
        qxa = plsc.load_gather(x_v, [qidxa])
        qya = plsc.load_gather(y_v, [qidxa])
        qza = plsc.load_gather(z_v, [qidxa])
        qxb = plsc.load_gather(x_v, [qidxb])
        qyb = plsc.load_gather(y_v, [qidxb])
        qzb = plsc.load_gather(z_v, [qidxb])

        def a_body(c, mins):
            mina, minb = mins
            base = c * _L
            cx = x_v[pl.ds(base, _L)]
            cy = y_v[pl.ds(base, _L)]
            cz = z_v[pl.ds(base, _L)]
            dxa = cx - qxa
            dya = cy - qya
            dza = cz - qza
            d2a = dxa * dxa + dya * dya + dza * dza
            dxb = cx - qxb
            dyb = cy - qyb
            dzb = cz - qzb
            d2b = dxb * dxb + dyb * dyb + dzb * dzb
            d2a_v[pl.ds(base, _L)] = d2a
            d2b_v[pl.ds(base, _L)] = d2b
            return (jnp.minimum(mina, d2a), jnp.minimum(minb, d2b))

        mina, minb = lax.fori_loop(0, _NCHUNK, a_body, (inf_v, inf_v),
                                   unroll=4)
        ta = jnp.zeros((_L,), jnp.float32) + jnp.max(mina)
        tb = jnp.zeros((_L,), jnp.float32) + jnp.max(minb)

        lanebase = lanes * _NCHUNK

        def b_body(c, cur):
            pcura, pcurb = cur
            base = c * _L
            d2a = d2a_v[pl.ds(base, _L)]
            d2b = d2b_v[pl.ds(base, _L)]
            ci = lanes + base
            ma = d2a <= ta
            mb = d2b <= tb
            desta = lanebase + pcura
            destb = lanebase + pcurb
            plsc.store_scatter(bufda_v, [desta], d2a, mask=ma)
            plsc.store_scatter(bufia_v, [desta], ci, mask=ma)
            plsc.store_scatter(bufdb_v, [destb], d2b, mask=mb)
            plsc.store_scatter(bufib_v, [destb], ci, mask=mb)
            return (pcura + jnp.where(ma, 1, 0), pcurb + jnp.where(mb, 1, 0))

        pcura, pcurb = lax.fori_loop(0, _NCHUNK, b_body, (zero_i, zero_i),
                                     unroll=4)

        def _select(pcur, bufd, bufi):
            rows = jnp.max(pcur)

            def s_body(r, bst):
                bd, bi = bst
                valid = r < pcur
                src = lanebase + r
                cd = plsc.load_gather(bufd, [src])
                cd = jnp.where(valid, cd, jnp.inf)
                ci2 = plsc.load_gather(bufi, [src])
                return _mergev(bd, bi, cd, ci2)

            return lax.fori_loop(0, rows, s_body, (inf_v, zero_i))

        bda, bia = _select(pcura, bufda_v, bufia_v)
        bdb, bib = _select(pcurb, bufdb_v, bufib_v)

        inv = jnp.float32(1.0 / _K)
        for q, qi2, bi2, qx, qy, qz in (
                (qa, 2 * qi, bia, qxa, qya, qza),
                (qb, 2 * qi + 1, bib, qxb, qyb, qzb)):
            nx = plsc.load_gather(x_v, [bi2])
            ny = plsc.load_gather(y_v, [bi2])
            nz = plsc.load_gather(z_v, [bi2])
            cxs = jnp.sum(nx) * inv - qx[0]
            cys = jnp.sum(ny) * inv - qy[0]
            czs = jnp.sum(nz) * inv - qz[0]
            vals = jnp.where(lanes == 0, cxs, jnp.where(lanes == 1, cys, czs))
            idxv = qi2 + _QPW * jnp.minimum(lanes, 2)
            plsc.store_scatter(ctx_v, [idxv], vals, mask=lanes < 3)
        return carry

    lax.fori_loop(0, _QPW // 2, q_body, 0)
    pltpu.sync_copy(ctx_v, out_hbm.at[pl.ds(wid * 3 * _QPW, 3 * _QPW)])


_knn_sc = functools.partial(
    pl.kernel,
    out_type=jax.ShapeDtypeStruct((_NW * 3 * _QPW,), jnp.float32),
    mesh=plsc.VectorSubcoreMesh(core_axis_name="c", subcore_axis_name="s"),
    compiler_params=pltpu.CompilerParams(needs_layout_passes=False),
    scratch_types=[
        pltpu.VMEM((_N,), jnp.float32),
        pltpu.VMEM((_N,), jnp.float32),
        pltpu.VMEM((_N,), jnp.float32),
        pltpu.VMEM((3 * _QPW,), jnp.float32),
        pltpu.VMEM((_N,), jnp.float32),
        pltpu.VMEM((_N,), jnp.int32),
        pltpu.VMEM((_N,), jnp.float32),
        pltpu.VMEM((_N,), jnp.int32),
        pltpu.VMEM((_N,), jnp.float32),
        pltpu.VMEM((_N,), jnp.float32),
    ],
)(_knn_body)


def _mlp_body(x_ref, goal_ref, w1_ref, b1_ref, w2_ref, b2_ref, wg_ref,
              bg_ref, wb_ref, bb_ref, wa_ref, ba_ref, o_ref):
    x = x_ref[...]
    h = jnp.maximum(
        jnp.dot(x, w1_ref[...], preferred_element_type=jnp.float32)
        + b1_ref[...], 0.0)
    f = jnp.maximum(
        jnp.dot(h, w2_ref[...], preferred_element_type=jnp.float32)
        + b2_ref[...], 0.0)
    goal = goal_ref[0]
    g = jnp.dot(goal, wg_ref[...],
                preferred_element_type=jnp.float32) + bg_ref[...]
    bt = jnp.dot(goal, wb_ref[...],
                 preferred_element_type=jnp.float32) + bb_ref[...]
    f = g * f + bt
    o_ref[...] = jnp.maximum(
        jnp.dot(f, wa_ref[...], preferred_element_type=jnp.float32)
        + ba_ref[...], 0.0)


_ROWS = 512
_GOAL_DIM = 16
_HID1 = 64
_HID2 = 128
_AFF = 32


def _mlp_tc(x, goal, w1t, b1, w2t, b2, wgt, bg, wbt, bb, wat, ba):
    nblk = _B * _N // _ROWS
    blk_per_b = _N // _ROWS
    rep = lambda i: (0, 0)
    return pl.pallas_call(
        _mlp_body,
        grid=(nblk,),
        in_specs=[
            pl.BlockSpec((_ROWS, 8), lambda i: (i, 0)),
            pl.BlockSpec((1, 1, _GOAL_DIM), lambda i: (i // blk_per_b, 0, 0)),
            pl.BlockSpec((8, _HID1), rep),
            pl.BlockSpec((1, _HID1), rep),
            pl.BlockSpec((_HID1, _HID2), rep),
            pl.BlockSpec((1, _HID2), rep),
            pl.BlockSpec((_GOAL_DIM, _HID2), rep),
            pl.BlockSpec((1, _HID2), rep),
            pl.BlockSpec((_GOAL_DIM, _HID2), rep),
            pl.BlockSpec((1, _HID2), rep),
            pl.BlockSpec((_HID2, _AFF), rep),
            pl.BlockSpec((1, _AFF), rep),
        ],
        out_specs=pl.BlockSpec((_ROWS, _AFF), lambda i: (i, 0)),
        out_shape=jax.ShapeDtypeStruct((_B * _N, _AFF), jnp.float32),
    )(x, goal, w1t, b1, w2t, b2, wgt, bg, wbt, bb, wat, ba)


def kernel(pos, goal, W1, b1, W2, b2, Wg, bg, Wb, bb, Wa, ba):
    posT = jnp.transpose(pos, (0, 2, 1)).reshape(-1)
    ctx = _knn_sc(posT)
    ctx = (ctx.reshape(_B, _WPB, 3, _QPW)
              .transpose(0, 2, 1, 3)
              .reshape(_B, 3, _N)
              .transpose(0, 2, 1))
    x = jnp.concatenate(
        [pos, ctx, jnp.zeros((_B, _N, 2), jnp.float32)], axis=-1
    ).reshape(_B * _N, 8)
    w1t = jnp.pad(W1.T, ((0, 2), (0, 0)))
    out = _mlp_tc(x, goal.reshape(_B, 1, _GOAL_DIM), w1t,
                  b1.reshape(1, -1), W2.T, b2.reshape(1, -1),
                  Wg.T, bg.reshape(1, -1), Wb.T, bb.reshape(1, -1),
                  Wa.T, ba.reshape(1, -1))
    return out.reshape(_B, _N, _AFF)


# speedup vs baseline: 2.2909x; 2.2909x over previous
import functools

import jax
import jax.numpy as jnp
from jax import lax
from jax.experimental import pallas as pl
from jax.experimental.pallas import tpu as pltpu
from jax.experimental.pallas import tpu_sc as plsc

_B, _N, _D = 4, 4096, 3
_K = 16
_L = 16
_NC, _NS = 2, 16
_NW = _NC * _NS
_QPW = _B * _N // _NW
_WPB = _NW // _B
_NCHUNK = _N // _L
_QI = 2


def _knn_body(posT_hbm, out_hbm, x_v, y_v, z_v, ctx_v,
              bufda_v, bufia_v, bufdb_v, bufib_v, d2a_v, d2b_v):
    cid = lax.axis_index("c")
    sid = lax.axis_index("s")
    wid = cid * _NS + sid
    b = wid // _WPB
    qoff = (wid % _WPB) * _QPW
    pltpu.sync_copy(posT_hbm.at[pl.ds((b * 3 + 0) * _N, _N)], x_v)
    pltpu.sync_copy(posT_hbm.at[pl.ds((b * 3 + 1) * _N, _N)], y_v)
    pltpu.sync_copy(posT_hbm.at[pl.ds((b * 3 + 2) * _N, _N)], z_v)

    lanes = lax.iota(jnp.int32, _L)

    inf_v = jnp.full((_L,), jnp.inf, jnp.float32)
    zero_i = jnp.zeros((_L,), jnp.int32)

    def _mergev(bd, bi, cd, ci):
        sd, si = plsc.sort_key_val(cd, ci)
        rd = lax.rev(sd, (0,))
        ri = lax.rev(si, (0,))
        keep = bd <= rd
        nd = jnp.where(keep, bd, rd)
        ni = jnp.where(keep, bi, ri)
        nd2, ni2 = plsc.sort_key_val(nd, ni)
        return nd2, ni2

    def q_body(qi, carry):
        qa = qoff + 2 * qi
        qb = qa + 1
        qidxa = zero_i + qa
        qidxb = zero_i + qb
        qxa = plsc.load_gather(x_v, [qidxa])
        qya = plsc.load_gather(y_v, [qidxa])
        qza = plsc.load_gather(z_v, [qidxa])
        qxb = plsc.load_gather(x_v, [qidxb])
        qyb = plsc.load_gather(y_v, [qidxb])
        qzb = plsc.load_gather(z_v, [qidxb])

        def a_body(c, mins):
            mina, minb = mins
            base = c * _L
            cx = x_v[pl.ds(base, _L)]
            cy = y_v[pl.ds(base, _L)]
            cz = z_v[pl.ds(base, _L)]
            dxa = cx - qxa
            dya = cy - qya
            dza = cz - qza
            d2a = dxa * dxa + dya * dya + dza * dza
            dxb = cx - qxb
            dyb = cy - qyb
            dzb = cz - qzb
            d2b = dxb * dxb + dyb * dyb + dzb * dzb
            d2a_v[pl.ds(base, _L)] = d2a
            d2b_v[pl.ds(base, _L)] = d2b
            return (jnp.minimum(mina, d2a), jnp.minimum(minb, d2b))

        mina, minb = plsc.parallel_loop(
            0, _NCHUNK, carry=(inf_v, inf_v), unroll=4)(a_body)
        ta = jnp.zeros((_L,), jnp.float32) + jnp.max(mina)
        tb = jnp.zeros((_L,), jnp.float32) + jnp.max(minb)

        def b_body(c, cur):
            cva, cvb = cur
            base = c * 2 * _L
            d2a0 = d2a_v[pl.ds(base, _L)]
            d2b0 = d2b_v[pl.ds(base, _L)]
            d2a1 = d2a_v[pl.ds(base + _L, _L)]
            d2b1 = d2b_v[pl.ds(base + _L, _L)]
            ci0 = lanes + base
            ci1 = ci0 + _L
            ma0 = d2a0 <= ta
            mb0 = d2b0 <= tb
            ma1 = d2a1 <= ta
            mb1 = d2b1 <= tb
            pca0 = plsc.all_reduce_population_count(ma0)
            pcb0 = plsc.all_reduce_population_count(mb0)
            pca1 = plsc.all_reduce_population_count(ma1)
            pcb1 = plsc.all_reduce_population_count(mb1)
            packed = (jnp.where(ma0, 1, 0) + jnp.where(mb0, 1 << 8, 0)
                      + jnp.where(ma1, 1 << 16, 0)
                      + jnp.where(mb1, 1 << 24, 0))
            cs = plsc.cumsum(packed)
            desta0 = cva + (cs & 255) - 1
            destb0 = cvb + ((cs >> 8) & 255) - 1
            cva1 = cva + pca0
            cvb1 = cvb + pcb0
            desta1 = cva1 + ((cs >> 16) & 255) - 1
            destb1 = cvb1 + ((cs >> 24) & 255) - 1
            plsc.store_scatter(bufda_v, [desta0], d2a0, mask=ma0)
            plsc.store_scatter(bufia_v, [desta0], ci0, mask=ma0)
            plsc.store_scatter(bufdb_v, [destb0], d2b0, mask=mb0)
            plsc.store_scatter(bufib_v, [destb0], ci0, mask=mb0)
            plsc.store_scatter(bufda_v, [desta1], d2a1, mask=ma1)
            plsc.store_scatter(bufia_v, [desta1], ci1, mask=ma1)
            plsc.store_scatter(bufdb_v, [destb1], d2b1, mask=mb1)
            plsc.store_scatter(bufib_v, [destb1], ci1, mask=mb1)
            return (cva1 + pca1, cvb1 + pcb1)

        cva, cvb = plsc.parallel_loop(
            0, _NCHUNK // 2, carry=(zero_i, zero_i), unroll=2)(b_body)

        def _select(cnt_vec, bufd, bufi):
            cnt = cnt_vec[0]
            nch = (cnt + _L - 1) // _L

            def s_body(j, bst):
                bd, bi = bst
                valid = lanes < (cnt - j * _L)
                cd = jnp.where(valid, bufd[pl.ds(j * _L, _L)], jnp.inf)
                ci2 = bufi[pl.ds(j * _L, _L)]
                return _mergev(bd, bi, cd, ci2)

            return lax.fori_loop(0, nch, s_body, (inf_v, zero_i))

        bda, bia = _select(cva, bufda_v, bufia_v)
        bdb, bib = _select(cvb, bufdb_v, bufib_v)

        inv = jnp.float32(1.0 / _K)
        for q, qi2, bi2, qx, qy, qz in (
                (qa, 2 * qi, bia, qxa, qya, qza),
                (qb, 2 * qi + 1, bib, qxb, qyb, qzb)):
            nx = plsc.load_gather(x_v, [bi2])
            ny = plsc.load_gather(y_v, [bi2])
            nz = plsc.load_gather(z_v, [bi2])
            cxs = jnp.sum(nx) * inv - qx[0]
            cys = jnp.sum(ny) * inv - qy[0]
            czs = jnp.sum(nz) * inv - qz[0]
            vals = jnp.where(lanes == 0, cxs, jnp.where(lanes == 1, cys, czs))
            idxv = qi2 + _QPW * jnp.minimum(lanes, 2)
            plsc.store_scatter(ctx_v, [idxv], vals, mask=lanes < 3)
        return carry

    lax.fori_loop(0, _QPW // 2, q_body, 0)
    pltpu.sync_copy(ctx_v, out_hbm.at[pl.ds(wid * 3 * _QPW, 3 * _QPW)])


_knn_sc = functools.partial(
    pl.kernel,
    out_type=jax.ShapeDtypeStruct((_NW * 3 * _QPW,), jnp.float32),
    mesh=plsc.VectorSubcoreMesh(core_axis_name="c", subcore_axis_name="s"),
    compiler_params=pltpu.CompilerParams(needs_layout_passes=False),
    scratch_types=[
        pltpu.VMEM((_N,), jnp.float32),
        pltpu.VMEM((_N,), jnp.float32),
        pltpu.VMEM((_N,), jnp.float32),
        pltpu.VMEM((3 * _QPW,), jnp.float32),
        pltpu.VMEM((_N,), jnp.float32),
        pltpu.VMEM((_N,), jnp.int32),
        pltpu.VMEM((_N,), jnp.float32),
        pltpu.VMEM((_N,), jnp.int32),
        pltpu.VMEM((_N,), jnp.float32),
        pltpu.VMEM((_N,), jnp.float32),
    ],
)(_knn_body)


def _mlp_body(x_ref, goal_ref, w1_ref, b1_ref, w2_ref, b2_ref, wg_ref,
              bg_ref, wb_ref, bb_ref, wa_ref, ba_ref, o_ref):
    x = x_ref[...]
    h = jnp.maximum(
        jnp.dot(x, w1_ref[...], preferred_element_type=jnp.float32)
        + b1_ref[...], 0.0)
    f = jnp.maximum(
        jnp.dot(h, w2_ref[...], preferred_element_type=jnp.float32)
        + b2_ref[...], 0.0)
    goal = goal_ref[0]
    g = jnp.dot(goal, wg_ref[...],
                preferred_element_type=jnp.float32) + bg_ref[...]
    bt = jnp.dot(goal, wb_ref[...],
                 preferred_element_type=jnp.float32) + bb_ref[...]
    f = g * f + bt
    o_ref[...] = jnp.maximum(
        jnp.dot(f, wa_ref[...], preferred_element_type=jnp.float32)
        + ba_ref[...], 0.0)


_ROWS = 512
_GOAL_DIM = 16
_HID1 = 64
_HID2 = 128
_AFF = 32


def _mlp_tc(x, goal, w1t, b1, w2t, b2, wgt, bg, wbt, bb, wat, ba):
    nblk = _B * _N // _ROWS
    blk_per_b = _N // _ROWS
    rep = lambda i: (0, 0)
    return pl.pallas_call(
        _mlp_body,
        grid=(nblk,),
        in_specs=[
            pl.BlockSpec((_ROWS, 8), lambda i: (i, 0)),
            pl.BlockSpec((1, 1, _GOAL_DIM), lambda i: (i // blk_per_b, 0, 0)),
            pl.BlockSpec((8, _HID1), rep),
            pl.BlockSpec((1, _HID1), rep),
            pl.BlockSpec((_HID1, _HID2), rep),
            pl.BlockSpec((1, _HID2), rep),
            pl.BlockSpec((_GOAL_DIM, _HID2), rep),
            pl.BlockSpec((1, _HID2), rep),
            pl.BlockSpec((_GOAL_DIM, _HID2), rep),
            pl.BlockSpec((1, _HID2), rep),
            pl.BlockSpec((_HID2, _AFF), rep),
            pl.BlockSpec((1, _AFF), rep),
        ],
        out_specs=pl.BlockSpec((_ROWS, _AFF), lambda i: (i, 0)),
        out_shape=jax.ShapeDtypeStruct((_B * _N, _AFF), jnp.float32),
    )(x, goal, w1t, b1, w2t, b2, wgt, bg, wbt, bb, wat, ba)


def kernel(pos, goal, W1, b1, W2, b2, Wg, bg, Wb, bb, Wa, ba):
    posT = jnp.transpose(pos, (0, 2, 1)).reshape(-1)
    ctx = _knn_sc(posT)
    ctx = (ctx.reshape(_B, _WPB, 3, _QPW)
              .transpose(0, 2, 1, 3)
              .reshape(_B, 3, _N)
              .transpose(0, 2, 1))
    x = jnp.concatenate(
        [pos, ctx, jnp.zeros((_B, _N, 2), jnp.float32)], axis=-1
    ).reshape(_B * _N, 8)
    w1t = jnp.pad(W1.T, ((0, 2), (0, 0)))
    out = _mlp_tc(x, goal.reshape(_B, 1, _GOAL_DIM), w1t,
                  b1.reshape(1, -1), W2.T, b2.reshape(1, -1),
                  Wg.T, bg.reshape(1, -1), Wb.T, bb.reshape(1, -1),
                  Wa.T, ba.reshape(1, -1))
    return out.reshape(_B, _N, _AFF)
